# baseline (speedup 1.0000x reference)
import functools

import jax
import jax.numpy as jnp
from jax import lax
from jax.experimental import pallas as pl
from jax.experimental.pallas import tpu as pltpu
from jax.experimental.pallas import tpu_sc as plsc

BATCH = 16384
EMBED_DIM = 32
NUM_CHUNKS = 4
CHUNK = 128
PER_WORKER = NUM_CHUNKS * CHUNK


def _sc_body(uid_hbm, iid_hbm, utab_hbm, itab_hbm, out_hbm,
             uidx_v, iidx_v, u_rows, v_rows, out_v, sem_u, sem_v):
    nc = 2
    wid = lax.axis_index("s") * nc + lax.axis_index("c")

    pltpu.sync_copy(uid_hbm.at[wid], uidx_v)
    pltpu.sync_copy(iid_hbm.at[wid], iidx_v)

    copies = []
    for j in range(NUM_CHUNKS):
        copies.append(
            pltpu.async_copy(utab_hbm.at[uidx_v.at[j]], u_rows.at[j], sem_u))
        copies.append(
            pltpu.async_copy(itab_hbm.at[iidx_v.at[j]], v_rows.at[j], sem_v))
    for c in copies:
        c.wait()

    lanes = lax.iota(jnp.int32, 16)
    perms = [jnp.bitwise_xor(lanes, k) for k in (8, 4, 2, 1)]

    def group_body(j, g):
        def row_body(r, acc):
            row = g * 16 + r
            u0 = u_rows[j, row, pl.ds(0, 16)]
            u1 = u_rows[j, row, pl.ds(16, 16)]
            v0 = v_rows[j, row, pl.ds(0, 16)]
            v1 = v_rows[j, row, pl.ds(16, 16)]
            t = u0 * v0 + u1 * v1
            for p in perms:
                t = t + t.at[p].get(mode="promise_in_bounds",
                                    unique_indices=True)
            return jnp.where(lanes == r, t, acc)

        acc = lax.fori_loop(0, 16, row_body, jnp.zeros((16,), jnp.float32))
        out_v[j, pl.ds(g * 16, 16)] = acc

    for j in range(NUM_CHUNKS):
        lax.fori_loop(0, CHUNK // 16,
                      lambda g, _, j=j: (group_body(j, g), 0)[1], 0)

    pltpu.sync_copy(out_v, out_hbm.at[wid])


@jax.jit
def _mf_dot(user_ids, item_ids, user_table, item_table):
    mesh = plsc.VectorSubcoreMesh(core_axis_name="c", subcore_axis_name="s")
    f = functools.partial(
        pl.kernel,
        mesh=mesh,
        compiler_params=pltpu.CompilerParams(use_tc_tiling_on_sc=False),
        out_type=jax.ShapeDtypeStruct((32, NUM_CHUNKS, CHUNK), jnp.float32),
        scratch_types=[
            pltpu.VMEM((NUM_CHUNKS, CHUNK), jnp.int32),
            pltpu.VMEM((NUM_CHUNKS, CHUNK), jnp.int32),
            pltpu.VMEM((NUM_CHUNKS, CHUNK, EMBED_DIM), jnp.float32),
            pltpu.VMEM((NUM_CHUNKS, CHUNK, EMBED_DIM), jnp.float32),
            pltpu.VMEM((NUM_CHUNKS, CHUNK), jnp.float32),
            pltpu.SemaphoreType.DMA,
            pltpu.SemaphoreType.DMA,
        ],
    )(_sc_body)
    return f(user_ids, item_ids, user_table, item_table)


def kernel(X, user_table, item_table):
    user_ids = X[:, 0].reshape(32, NUM_CHUNKS, CHUNK)
    item_ids = X[:, 1].reshape(32, NUM_CHUNKS, CHUNK)
    out = _mf_dot(user_ids, item_ids, user_table, item_table)
    return out.reshape(BATCH, 1)
